# Initial kernel scaffold; baseline (speedup 1.0000x reference)
import jax
import jax.numpy as jnp
from jax.experimental import pallas as pl


def kernel(v):
    raise NotImplementedError("write your pallas kernel here")


# speedup vs baseline: 59.0970x; 59.0970x over previous
import jax
import jax.numpy as jnp
from jax.experimental import pallas as pl
from jax.experimental.pallas import tpu as pltpu

_L = 8192
_DV = 1024
_LANES = 256


def _swd_block(x_ref, o_ref, *, lane_base):
    x = x_ref[0]
    up = pltpu.roll(x, x.shape[0] - 1, 0)
    dn = pltpu.roll(x, 1, 0)
    row = jax.lax.broadcasted_iota(jnp.int32, x.shape, 0)
    even = (row & 1) == 0
    w = jnp.where(even, jnp.minimum(x, up), jnp.maximum(x, dn))
    j = lane_base + jax.lax.broadcasted_iota(jnp.int32, (1, x.shape[1]), 1)
    s = 2 * jnp.maximum(j - 1, 0)
    for b in range(1, 11):
        take = ((s >> b) & 1) == 1
        w = jnp.where(take, pltpu.roll(w, 1 << b, 0), w)
    o_ref[0] = w


def _index_map(b, g):
    return (b, 0, g)


def kernel(v):
    batch, L, dv = v.shape
    grid = (batch, dv // _LANES)

    def body(x_ref, o_ref):
        g = pl.program_id(1)
        _swd_block(x_ref, o_ref, lane_base=g * _LANES)

    return pl.pallas_call(
        body,
        grid=grid,
        in_specs=[pl.BlockSpec((1, L, _LANES), _index_map)],
        out_specs=pl.BlockSpec((1, L, _LANES), _index_map),
        out_shape=jax.ShapeDtypeStruct(v.shape, v.dtype),
    )(v)
